# baseline (speedup 1.0000x reference)
import functools

import jax
import jax.numpy as jnp
from jax import lax
from jax.experimental import pallas as pl
from jax.experimental.pallas import tpu as pltpu
from jax.experimental.pallas import tpu_sc as plsc

_B = 16384
_K = 64

_info = plsc.get_sparse_core_info()
_NC = _info.num_cores
_NS = _info.num_subcores
_NW = _NC * _NS
_BPW = _B // _NW

_mesh = plsc.VectorSubcoreMesh(core_axis_name="c", subcore_axis_name="s")


@functools.partial(
    pl.kernel,
    mesh=_mesh,
    compiler_params=pltpu.CompilerParams(use_tc_tiling_on_sc=False),
    out_type=[
        jax.ShapeDtypeStruct((_B, _K), jnp.float32),
        jax.ShapeDtypeStruct((_B, _K), jnp.float32),
    ],
    scratch_types=[
        pltpu.VMEM((_BPW,), jnp.int32),
        pltpu.VMEM((_BPW, _K), jnp.float32),
        pltpu.VMEM((_BPW,), jnp.int32),
        pltpu.VMEM((_BPW, _K), jnp.float32),
        pltpu.SemaphoreType.DMA,
        pltpu.SemaphoreType.DMA,
    ],
)
def _gather_pair(
    gu_hbm,
    gi_hbm,
    users_hbm,
    items_hbm,
    out_u_hbm,
    out_i_hbm,
    uidx_v,
    urows_v,
    iidx_v,
    irows_v,
    sem_u,
    sem_i,
):
    wid = lax.axis_index("s") * _NC + lax.axis_index("c")
    base = wid * _BPW
    pltpu.sync_copy(users_hbm.at[pl.ds(base, _BPW)], uidx_v)
    pltpu.sync_copy(items_hbm.at[pl.ds(base, _BPW)], iidx_v)
    cu = pltpu.async_copy(gu_hbm.at[uidx_v], urows_v, sem_u)
    ci = pltpu.async_copy(gi_hbm.at[iidx_v], irows_v, sem_i)
    cu.wait()
    pltpu.sync_copy(urows_v, out_u_hbm.at[pl.ds(base, _BPW)])
    ci.wait()
    pltpu.sync_copy(irows_v, out_i_hbm.at[pl.ds(base, _BPW)])


def kernel(Gu, Gi, users, items):
    out_u, out_i = _gather_pair(
        Gu, Gi, users.astype(jnp.int32), items.astype(jnp.int32)
    )
    return (out_u, out_i)
